# Initial kernel scaffold; baseline (speedup 1.0000x reference)
import jax
import jax.numpy as jnp
from jax.experimental import pallas as pl


def kernel(sequences, durations, max_mel_length):
    raise NotImplementedError("write your pallas kernel here")


# speedup vs baseline: 10.0221x; 10.0221x over previous
import functools

import jax
import jax.numpy as jnp
from jax import lax
from jax.experimental import pallas as pl
from jax.experimental.pallas import tpu as pltpu
from jax.experimental.pallas import tpu_sc as plsc

B, T, D = 16, 512, 256
L = 2048
LANES = 16
NTOK_CH = T // LANES
NFRM_CH = L // LANES
ROWS = 128
HALF = L // 2
N_CHUNKS = HALF // ROWS


def _lr_body(table, dur, out, d_out, dur_v, d_v, z_v, gidx_v, rows_v, zero_v,
             sem):
    b = lax.axis_index("s")
    h = lax.axis_index("c")

    pltpu.sync_copy(dur.at[b], dur_v)

    zeros16i = jnp.zeros((LANES,), jnp.int32)
    for i in range(NFRM_CH):
        z_v[pl.ds(i * LANES, LANES)] = zeros16i

    carry = jnp.int32(0)
    ids0 = lax.broadcasted_iota(jnp.int32, (LANES,), 0)
    for i in range(NTOK_CH):
        dv = dur_v[pl.ds(i * LANES, LANES)]
        d16 = jnp.maximum(dv, 1)
        d_v[pl.ds(i * LANES, LANES)] = d16
        cum16 = plsc.cumsum(d16) + carry
        starts = cum16 - d16
        carry = jnp.max(cum16)
        mask = starts < L
        starts_c = jnp.minimum(starts, L - 1)
        plsc.store_scatter(z_v, [starts_c], ids0 + (i * LANES), mask=mask)
    total = carry

    @pl.when(h == 0)
    def _():
        pltpu.sync_copy(d_v, d_out.at[b])

    mcarry = jnp.int32(0)
    base_row = b * T
    for i in range(NFRM_CH):
        zc = z_v[pl.ds(i * LANES, LANES)]
        m = jnp.maximum(plsc.cummax(zc), mcarry)
        mcarry = jnp.max(m)
        gidx_v[pl.ds(i * LANES, LANES)] = m + base_row

    zeros16f = jnp.zeros((LANES,), jnp.float32)

    def _zero_rows(ref, lo, hi):
        def body(r, _):
            for k in range(D // LANES):
                ref[r, pl.ds(k * LANES, LANES)] = zeros16f
            return 0
        lax.fori_loop(lo, hi, body, 0)

    _zero_rows(zero_v, 0, ROWS)

    half_base = h * HALF
    for cch in range(N_CHUNKS):
        fb = pl.multiple_of(half_base + cch * ROWS, ROWS)
        live = jnp.clip(total - (half_base + cch * ROWS), 0, ROWS)

        @pl.when(live > 0)
        def _():
            pltpu.async_copy(table.at[gidx_v.at[pl.ds(fb, ROWS)]], rows_v,
                             sem).wait()

            @pl.when(live < ROWS)
            def _():
                _zero_rows(rows_v, live, ROWS)

            pltpu.sync_copy(rows_v, out.at[b, pl.ds(fb, ROWS)])

        @pl.when(live == 0)
        def _():
            pltpu.sync_copy(zero_v, out.at[b, pl.ds(fb, ROWS)])


def kernel(sequences, durations, max_mel_length):
    table = sequences.reshape(B * T, D)
    mesh = plsc.VectorSubcoreMesh(core_axis_name="c", subcore_axis_name="s")
    run = functools.partial(
        pl.kernel,
        mesh=mesh,
        compiler_params=pltpu.CompilerParams(needs_layout_passes=False),
        out_type=(jax.ShapeDtypeStruct((B, L, D), jnp.float32),
                  jax.ShapeDtypeStruct((B, T), jnp.int32)),
        scratch_types=[
            pltpu.VMEM((T,), jnp.int32),
            pltpu.VMEM((T,), jnp.int32),
            pltpu.VMEM((L,), jnp.int32),
            pltpu.VMEM((L,), jnp.int32),
            pltpu.VMEM((ROWS, D), jnp.float32),
            pltpu.VMEM((ROWS, D), jnp.float32),
            pltpu.SemaphoreType.DMA,
        ],
    )(_lr_body)
    out, d = run(table, durations)
    return out, d


# speedup vs baseline: 10.6807x; 1.0657x over previous
import functools

import jax
import jax.numpy as jnp
from jax import lax
from jax.experimental import pallas as pl
from jax.experimental.pallas import tpu as pltpu
from jax.experimental.pallas import tpu_sc as plsc

B, T, D = 16, 512, 256
L = 2048
LANES = 16
NTOK_CH = T // LANES
NFRM_CH = L // LANES
ROWS = 128
HALF = L // 2
N_CHUNKS = HALF // ROWS


def _lr_body(table, dur, out, d_out, dur_v, d_v, z_v, gidx_v, rows_v, zero_v,
             gsem0, gsem1, wsem0, wsem1):
    gsem = (gsem0, gsem1)
    wsem = (wsem0, wsem1)
    b = lax.axis_index("s")
    h = lax.axis_index("c")

    pltpu.sync_copy(dur.at[b], dur_v)

    zeros16i = jnp.zeros((LANES,), jnp.int32)
    for i in range(NFRM_CH):
        z_v[pl.ds(i * LANES, LANES)] = zeros16i

    carry = jnp.int32(0)
    ids0 = lax.broadcasted_iota(jnp.int32, (LANES,), 0)
    for i in range(NTOK_CH):
        dv = dur_v[pl.ds(i * LANES, LANES)]
        d16 = jnp.maximum(dv, 1)
        d_v[pl.ds(i * LANES, LANES)] = d16
        cum16 = plsc.cumsum(d16) + carry
        starts = cum16 - d16
        carry = jnp.max(cum16)
        mask = starts < L
        starts_c = jnp.minimum(starts, L - 1)
        plsc.store_scatter(z_v, [starts_c], ids0 + (i * LANES), mask=mask)
    total = carry

    @pl.when(h == 0)
    def _():
        pltpu.sync_copy(d_v, d_out.at[b])

    mcarry = jnp.int32(0)
    base_row = b * T
    for i in range(NFRM_CH):
        zc = z_v[pl.ds(i * LANES, LANES)]
        m = jnp.maximum(plsc.cummax(zc), mcarry)
        mcarry = jnp.max(m)
        gidx_v[pl.ds(i * LANES, LANES)] = m + base_row

    zeros16f = jnp.zeros((LANES,), jnp.float32)

    def _zero_rows(ref, lo, hi):
        def body(r, _):
            for k in range(D // LANES):
                ref[r, pl.ds(k * LANES, LANES)] = zeros16f
            return 0
        lax.fori_loop(lo, hi, body, 0)

    _zero_rows(zero_v, 0, ROWS)

    half_base = h * HALF

    def _fb_live(cch):
        fb = pl.multiple_of(half_base + cch * ROWS, ROWS)
        live = jnp.clip(total - (half_base + cch * ROWS), 0, ROWS)
        return fb, live

    def _issue(cch):
        buf = cch % 2
        fb, live = _fb_live(cch)

        @pl.when(live > 0)
        def _():
            pltpu.async_copy(table.at[gidx_v.at[pl.ds(fb, ROWS)]],
                             rows_v.at[buf], gsem[buf])

    def _finish(cch):
        buf = cch % 2
        fb, live = _fb_live(cch)

        @pl.when(live > 0)
        def _():
            pltpu.make_async_copy(table.at[pl.ds(0, ROWS)], rows_v.at[buf],
                                  gsem[buf]).wait()

            @pl.when(live < ROWS)
            def _():
                _zero_rows(rows_v.at[buf], live, ROWS)

            pltpu.async_copy(rows_v.at[buf], out.at[b, pl.ds(fb, ROWS)],
                             wsem[buf])

        @pl.when(live == 0)
        def _():
            pltpu.async_copy(zero_v, out.at[b, pl.ds(fb, ROWS)], wsem[buf])

    def _drain_write(buf):
        pltpu.make_async_copy(table.at[pl.ds(0, ROWS)], rows_v.at[buf],
                              wsem[buf]).wait()

    _issue(0)
    _issue(1)
    for cch in range(N_CHUNKS):
        _finish(cch)
        if cch + 2 < N_CHUNKS:
            _drain_write(cch % 2)
            _issue(cch + 2)
    _drain_write(N_CHUNKS % 2)
    _drain_write((N_CHUNKS + 1) % 2)


def kernel(sequences, durations, max_mel_length):
    table = sequences.reshape(B * T, D)
    mesh = plsc.VectorSubcoreMesh(core_axis_name="c", subcore_axis_name="s")
    run = functools.partial(
        pl.kernel,
        mesh=mesh,
        compiler_params=pltpu.CompilerParams(needs_layout_passes=False),
        out_type=(jax.ShapeDtypeStruct((B, L, D), jnp.float32),
                  jax.ShapeDtypeStruct((B, T), jnp.int32)),
        scratch_types=[
            pltpu.VMEM((T,), jnp.int32),
            pltpu.VMEM((T,), jnp.int32),
            pltpu.VMEM((L,), jnp.int32),
            pltpu.VMEM((L,), jnp.int32),
            pltpu.VMEM((2, ROWS, D), jnp.float32),
            pltpu.VMEM((ROWS, D), jnp.float32),
            pltpu.SemaphoreType.DMA,
            pltpu.SemaphoreType.DMA,
            pltpu.SemaphoreType.DMA,
            pltpu.SemaphoreType.DMA,
        ],
    )(_lr_body)
    out, d = run(table, durations)
    return out, d


# speedup vs baseline: 10.7255x; 1.0042x over previous
import functools

import jax
import jax.numpy as jnp
from jax import lax
from jax.experimental import pallas as pl
from jax.experimental.pallas import tpu as pltpu
from jax.experimental.pallas import tpu_sc as plsc

B, T, D = 16, 512, 256
L = 2048
LANES = 16
NTOK_CH = T // LANES
NFRM_CH = L // LANES
ROWS = 128
HALF = L // 2
N_CHUNKS = HALF // ROWS


def _lr_body(table, dur, out, d_out, dur_v, d_v, z_v, gidx_v, rows_v, zero_v,
             gsem0, gsem1, wsem0, wsem1):
    gsem = (gsem0, gsem1)
    wsem = (wsem0, wsem1)
    b = lax.axis_index("s")
    h = lax.axis_index("c")

    pltpu.sync_copy(dur.at[b], dur_v)

    zeros16i = jnp.zeros((LANES,), jnp.int32)
    for i in range(NFRM_CH):
        z_v[pl.ds(i * LANES, LANES)] = zeros16i

    carry = jnp.int32(0)
    ids0 = lax.broadcasted_iota(jnp.int32, (LANES,), 0)
    for i in range(NTOK_CH):
        dv = dur_v[pl.ds(i * LANES, LANES)]
        d16 = jnp.maximum(dv, 1)
        d_v[pl.ds(i * LANES, LANES)] = d16
        cum16 = plsc.cumsum(d16) + carry
        starts = cum16 - d16
        carry = jnp.max(cum16)
        mask = starts < L
        starts_c = jnp.minimum(starts, L - 1)
        plsc.store_scatter(z_v, [starts_c], ids0 + (i * LANES), mask=mask)
    total = carry

    @pl.when(h == 0)
    def _():
        pltpu.sync_copy(d_v, d_out.at[b])

    mcarry = jnp.int32(0)
    base_row = b * T
    for i in range(NFRM_CH):
        zc = z_v[pl.ds(i * LANES, LANES)]
        m = jnp.maximum(plsc.cummax(zc), mcarry)
        mcarry = jnp.max(m)
        gidx_v[pl.ds(i * LANES, LANES)] = m + base_row

    zeros16f = jnp.zeros((LANES,), jnp.float32)

    def _zero_rows(ref, lo, hi):
        def body(r, _):
            for k in range(D // LANES):
                ref[r, pl.ds(k * LANES, LANES)] = zeros16f
            return 0
        lax.fori_loop(lo, hi, body, 0)

    _zero_rows(zero_v, 0, ROWS)


    def _fb_live(cch):
        start = (2 * cch + h) * ROWS
        fb = pl.multiple_of(start, ROWS)
        live = jnp.clip(total - start, 0, ROWS)
        return fb, live

    def _issue(cch):
        buf = cch % 2
        fb, live = _fb_live(cch)

        @pl.when(live > 0)
        def _():
            pltpu.async_copy(table.at[gidx_v.at[pl.ds(fb, ROWS)]],
                             rows_v.at[buf], gsem[buf])

    def _finish(cch):
        buf = cch % 2
        fb, live = _fb_live(cch)

        @pl.when(live > 0)
        def _():
            pltpu.make_async_copy(table.at[pl.ds(0, ROWS)], rows_v.at[buf],
                                  gsem[buf]).wait()

            @pl.when(live < ROWS)
            def _():
                _zero_rows(rows_v.at[buf], live, ROWS)

            pltpu.async_copy(rows_v.at[buf], out.at[b, pl.ds(fb, ROWS)],
                             wsem[buf])

        @pl.when(live == 0)
        def _():
            pltpu.async_copy(zero_v, out.at[b, pl.ds(fb, ROWS)], wsem[buf])

    def _drain_write(buf):
        pltpu.make_async_copy(table.at[pl.ds(0, ROWS)], rows_v.at[buf],
                              wsem[buf]).wait()

    _issue(0)
    _issue(1)
    for cch in range(N_CHUNKS):
        _finish(cch)
        if cch + 2 < N_CHUNKS:
            _drain_write(cch % 2)
            _issue(cch + 2)
    _drain_write(N_CHUNKS % 2)
    _drain_write((N_CHUNKS + 1) % 2)


def kernel(sequences, durations, max_mel_length):
    table = sequences.reshape(B * T, D)
    mesh = plsc.VectorSubcoreMesh(core_axis_name="c", subcore_axis_name="s")
    run = functools.partial(
        pl.kernel,
        mesh=mesh,
        compiler_params=pltpu.CompilerParams(needs_layout_passes=False),
        out_type=(jax.ShapeDtypeStruct((B, L, D), jnp.float32),
                  jax.ShapeDtypeStruct((B, T), jnp.int32)),
        scratch_types=[
            pltpu.VMEM((T,), jnp.int32),
            pltpu.VMEM((T,), jnp.int32),
            pltpu.VMEM((L,), jnp.int32),
            pltpu.VMEM((L,), jnp.int32),
            pltpu.VMEM((2, ROWS, D), jnp.float32),
            pltpu.VMEM((ROWS, D), jnp.float32),
            pltpu.SemaphoreType.DMA,
            pltpu.SemaphoreType.DMA,
            pltpu.SemaphoreType.DMA,
            pltpu.SemaphoreType.DMA,
        ],
    )(_lr_body)
    out, d = run(table, durations)
    return out, d
